# Initial kernel scaffold; baseline (speedup 1.0000x reference)
import functools

import jax
import jax.numpy as jnp
from jax import lax
from jax.experimental import pallas as pl
from jax.experimental.pallas import tpu as pltpu
from jax.experimental.pallas import tpu_sc as plsc

N_NODES = 10000
N_PAD = 10240
F = 256
FH = 128
OUT_F = 128
NC, NS = 2, 16
CHUNK = 128
AGG_CHUNKS = 80
DEG_CHUNKS = 40
E_PAD = NS * AGG_CHUNKS * CHUNK
ROWS_PER_TILE = N_PAD // NS
DEG_W = 16


def _deg_body(srcp, dstp, ones_hbm, zeros_hbm, out_hbm,
              src_v, dst_v, ones_v, stage_v, deg_o_s, deg_i_s):
    c = lax.axis_index("c")
    s = lax.axis_index("s")
    base = s * ROWS_PER_TILE
    pltpu.sync_copy(zeros_hbm, stage_v)
    pltpu.sync_copy(stage_v, deg_o_s.at[pl.ds(base, ROWS_PER_TILE)])
    pltpu.sync_copy(stage_v, deg_i_s.at[pl.ds(base, ROWS_PER_TILE)])
    pltpu.sync_copy(srcp.at[s, pl.ds(c * DEG_CHUNKS, DEG_CHUNKS)], src_v)
    pltpu.sync_copy(dstp.at[s, pl.ds(c * DEG_CHUNKS, DEG_CHUNKS)], dst_v)
    pltpu.sync_copy(ones_hbm, ones_v)
    plsc.subcore_barrier()

    def body(j, carry):
        pltpu.sync_copy(ones_v, deg_o_s.at[src_v.at[j]], add=True)
        pltpu.sync_copy(ones_v, deg_i_s.at[dst_v.at[j]], add=True)
        return carry

    lax.fori_loop(0, DEG_CHUNKS, body, 0)
    plsc.subcore_barrier()
    sl = pl.ds(base, ROWS_PER_TILE)
    pltpu.sync_copy(deg_o_s.at[sl], stage_v)

    @pl.when(c == 0)
    def _():
        pltpu.sync_copy(stage_v, out_hbm.at[0, 0, sl])

    @pl.when(c == 1)
    def _():
        pltpu.sync_copy(stage_v, out_hbm.at[1, 0, sl])

    pltpu.sync_copy(deg_i_s.at[sl], stage_v)

    @pl.when(c == 0)
    def _():
        pltpu.sync_copy(stage_v, out_hbm.at[0, 1, sl])

    @pl.when(c == 1)
    def _():
        pltpu.sync_copy(stage_v, out_hbm.at[1, 1, sl])


def _agg_body(h0, h1, srcp, dstp, zeros_hbm, out0, out1,
              src_v, dst_v, rows_v, agg_s):
    c = lax.axis_index("c")
    s = lax.axis_index("s")
    base = s * ROWS_PER_TILE
    pltpu.sync_copy(zeros_hbm, rows_v)
    for k in range(ROWS_PER_TILE // CHUNK):
        pltpu.sync_copy(rows_v, agg_s.at[pl.ds(base + k * CHUNK, CHUNK)])
    pltpu.sync_copy(srcp.at[s], src_v)
    pltpu.sync_copy(dstp.at[s], dst_v)
    plsc.subcore_barrier()

    def make_body(h_ref):
        def body(j, carry):
            pltpu.sync_copy(h_ref.at[src_v.at[j]], rows_v)
            pltpu.sync_copy(rows_v, agg_s.at[dst_v.at[j]], add=True)
            return carry
        return body

    @pl.when(c == 0)
    def _():
        lax.fori_loop(0, AGG_CHUNKS, make_body(h0), 0)

    @pl.when(c == 1)
    def _():
        lax.fori_loop(0, AGG_CHUNKS, make_body(h1), 0)

    plsc.subcore_barrier()
    for k in range(ROWS_PER_TILE // CHUNK):
        sl = pl.ds(base + k * CHUNK, CHUNK)
        pltpu.sync_copy(agg_s.at[sl], rows_v)

        @pl.when(c == 0)
        def _():
            pltpu.sync_copy(rows_v, out0.at[sl])

        @pl.when(c == 1)
        def _():
            pltpu.sync_copy(rows_v, out1.at[sl])


@functools.lru_cache(maxsize=None)
def _sc_calls():
    mesh = plsc.VectorSubcoreMesh(core_axis_name="c", subcore_axis_name="s",
                                  num_cores=NC, num_subcores=NS)
    deg = pl.kernel(
        _deg_body,
        out_type=jax.ShapeDtypeStruct((2, 2, N_PAD, DEG_W), jnp.float32),
        mesh=mesh,
        scratch_types=[
            pltpu.VMEM((DEG_CHUNKS, CHUNK), jnp.int32),
            pltpu.VMEM((DEG_CHUNKS, CHUNK), jnp.int32),
            pltpu.VMEM((CHUNK, DEG_W), jnp.float32),
            pltpu.VMEM((ROWS_PER_TILE, DEG_W), jnp.float32),
            pltpu.VMEM_SHARED((N_PAD, DEG_W), jnp.float32),
            pltpu.VMEM_SHARED((N_PAD, DEG_W), jnp.float32),
        ],
    )
    agg = pl.kernel(
        _agg_body,
        out_type=[jax.ShapeDtypeStruct((N_PAD, FH), jnp.float32),
                  jax.ShapeDtypeStruct((N_PAD, FH), jnp.float32)],
        mesh=mesh,
        scratch_types=[
            pltpu.VMEM((AGG_CHUNKS, CHUNK), jnp.int32),
            pltpu.VMEM((AGG_CHUNKS, CHUNK), jnp.int32),
            pltpu.VMEM((CHUNK, FH), jnp.float32),
            pltpu.VMEM_SHARED((N_PAD, FH), jnp.float32),
        ],
    )
    return deg, agg


_BR = 256


def _scale_out(deg_ref):
    deg_o = deg_ref[0, 0, :, 0] + deg_ref[1, 0, :, 0]
    return lax.rsqrt(jnp.maximum(deg_o, 1.0))[:, None]


def _scale_in(deg_ref):
    deg_i = deg_ref[0, 1, :, 0] + deg_ref[1, 1, :, 0]
    return lax.rsqrt(jnp.maximum(deg_i, 1.0))[:, None]


def _b1_kernel(x_ref, w_ref, deg_ref, out0_ref, out1_ref):
    h = jnp.dot(x_ref[:], w_ref[:], preferred_element_type=jnp.float32)
    h = h * _scale_out(deg_ref)
    out0_ref[:] = h[:, :FH]
    out1_ref[:] = h[:, FH:]


def _b2_kernel(a0_ref, a1_ref, deg_ref, b_ref, w_ref, out0_ref, out1_ref):
    s_i = _scale_in(deg_ref)
    s_o = _scale_out(deg_ref)
    y0 = jnp.maximum(a0_ref[:] * s_i + b_ref[0, :FH], 0.0) * s_o
    y1 = jnp.maximum(a1_ref[:] * s_i + b_ref[0, FH:], 0.0) * s_o
    h = (jnp.dot(y0, w_ref[:FH, :], preferred_element_type=jnp.float32)
         + jnp.dot(y1, w_ref[FH:, :], preferred_element_type=jnp.float32))
    out0_ref[:] = h[:, :FH]
    out1_ref[:] = h[:, FH:]


def _d_kernel(a0_ref, a1_ref, deg_ref, b_ref, wd_ref, bd_ref, out_ref, acc_ref):
    i = pl.program_id(0)

    @pl.when(i == 0)
    def _():
        acc_ref[:] = jnp.zeros_like(acc_ref)

    s_i = _scale_in(deg_ref)
    rows = i * _BR + lax.broadcasted_iota(jnp.int32, (_BR, 1), 0)
    mask = (rows < N_NODES).astype(jnp.float32)
    y0 = jnp.maximum(a0_ref[:] * s_i + b_ref[0, :FH], 0.0) * mask
    y1 = jnp.maximum(a1_ref[:] * s_i + b_ref[0, FH:], 0.0) * mask
    acc_ref[:, :FH] += jnp.sum(y0, axis=0, keepdims=True)
    acc_ref[:, FH:] += jnp.sum(y1, axis=0, keepdims=True)

    @pl.when(i == N_PAD // _BR - 1)
    def _():
        hg = acc_ref[:] * (1.0 / N_NODES)
        out_ref[:] = (jnp.dot(hg, wd_ref[:], preferred_element_type=jnp.float32)
                      + bd_ref[:])


@functools.lru_cache(maxsize=None)
def _tc_calls():
    nblk = N_PAD // _BR
    deg_spec = pl.BlockSpec((2, 2, _BR, DEG_W), lambda i: (0, 0, i, 0))
    half = pl.BlockSpec((_BR, FH), lambda i: (i, 0))
    full = pl.BlockSpec((_BR, F), lambda i: (i, 0))
    w_spec = pl.BlockSpec((F, F), lambda i: (0, 0))
    b_spec = pl.BlockSpec((1, F), lambda i: (0, 0))
    b1 = pl.pallas_call(
        _b1_kernel,
        grid=(nblk,),
        in_specs=[full, w_spec, deg_spec],
        out_specs=[half, half],
        out_shape=[jax.ShapeDtypeStruct((N_PAD, FH), jnp.float32)] * 2,
    )
    b2 = pl.pallas_call(
        _b2_kernel,
        grid=(nblk,),
        in_specs=[half, half, deg_spec, b_spec, w_spec],
        out_specs=[half, half],
        out_shape=[jax.ShapeDtypeStruct((N_PAD, FH), jnp.float32)] * 2,
    )
    d = pl.pallas_call(
        _d_kernel,
        grid=(nblk,),
        in_specs=[half, half, deg_spec, b_spec,
                  pl.BlockSpec((F, OUT_F), lambda i: (0, 0)),
                  pl.BlockSpec((1, OUT_F), lambda i: (0, 0))],
        out_specs=pl.BlockSpec((1, OUT_F), lambda i: (0, 0)),
        out_shape=jax.ShapeDtypeStruct((1, OUT_F), jnp.float32),
        scratch_shapes=[pltpu.VMEM((1, F), jnp.float32)],
    )
    return b1, b2, d


@jax.jit
def kernel(x, edge_index, W1, b1, W2, b2, Wd, bd):
    n_edges = edge_index.shape[1]
    src = edge_index[0].astype(jnp.int32)
    dst = edge_index[1].astype(jnp.int32)
    pad = jnp.full((E_PAD - n_edges,), N_NODES, jnp.int32)
    srcp = jnp.concatenate([src, pad]).reshape(NS, AGG_CHUNKS, CHUNK)
    dstp = jnp.concatenate([dst, pad]).reshape(NS, AGG_CHUNKS, CHUNK)
    xp = jnp.zeros((N_PAD, F), jnp.float32).at[:N_NODES].set(x)
    ones16 = jnp.ones((CHUNK, DEG_W), jnp.float32)
    zeros16 = jnp.zeros((ROWS_PER_TILE, DEG_W), jnp.float32)
    zeros128 = jnp.zeros((CHUNK, FH), jnp.float32)

    deg_call, agg_call = _sc_calls()
    b1_call, b2_call, d_call = _tc_calls()

    degp = deg_call(srcp, dstp, ones16, zeros16)
    h10, h11 = b1_call(xp, W1, degp)
    a10, a11 = agg_call(h10, h11, srcp, dstp, zeros128)
    h20, h21 = b2_call(a10, a11, degp, b1.reshape(1, F), W2)
    a20, a21 = agg_call(h20, h21, srcp, dstp, zeros128)
    out = d_call(a20, a21, degp, b2.reshape(1, F), Wd, bd.reshape(1, OUT_F))
    return out


# speedup vs baseline: 3.0321x; 3.0321x over previous
import functools

import jax
import jax.numpy as jnp
from jax import lax
from jax.experimental import pallas as pl
from jax.experimental.pallas import tpu as pltpu
from jax.experimental.pallas import tpu_sc as plsc

N_NODES = 10000
N_PAD = 10240
F = 256
FH = 128
OUT_F = 128
NC, NS = 2, 16
CHUNK = 128
AGG_CHUNKS = 80
DEG_CHUNKS = 40
E_PAD = NS * AGG_CHUNKS * CHUNK
ROWS_PER_TILE = N_PAD // NS
DEG_W = 16


def _deg_body(srcp, dstp, ones_hbm, zeros_hbm, out_hbm, idx_v, const_v, deg_s):
    c = lax.axis_index("c")
    s = lax.axis_index("s")
    base = s * ROWS_PER_TILE
    pltpu.sync_copy(zeros_hbm.at[pl.ds(0, ROWS_PER_TILE)],
                    deg_s.at[pl.ds(base, ROWS_PER_TILE)])

    @pl.when(c == 0)
    def _():
        pltpu.sync_copy(srcp.at[s], idx_v)

    @pl.when(c == 1)
    def _():
        pltpu.sync_copy(dstp.at[s], idx_v)

    pltpu.sync_copy(ones_hbm, const_v)
    plsc.subcore_barrier()

    def body(j, carry):
        pltpu.sync_copy(const_v, deg_s.at[idx_v.at[j]], add=True)
        return carry

    lax.fori_loop(0, AGG_CHUNKS, body, 0)
    plsc.subcore_barrier()
    sl = pl.ds(base, ROWS_PER_TILE)

    @pl.when(c == 0)
    def _():
        pltpu.sync_copy(deg_s.at[sl], out_hbm.at[0, sl])

    @pl.when(c == 1)
    def _():
        pltpu.sync_copy(deg_s.at[sl], out_hbm.at[1, sl])


def _agg_body(h0, h1, srcp, dstp, zeros_hbm, out0, out1,
              src_v, dst_v, rows_v, agg_s):
    c = lax.axis_index("c")
    s = lax.axis_index("s")
    base = s * ROWS_PER_TILE
    pltpu.sync_copy(zeros_hbm.at[pl.ds(0, ROWS_PER_TILE)],
                    agg_s.at[pl.ds(base, ROWS_PER_TILE)])
    pltpu.sync_copy(srcp.at[s], src_v)
    pltpu.sync_copy(dstp.at[s], dst_v)
    plsc.subcore_barrier()

    def make_body(h_ref):
        def body(j, carry):
            pltpu.sync_copy(h_ref.at[src_v.at[j]], rows_v)
            pltpu.sync_copy(rows_v, agg_s.at[dst_v.at[j]], add=True)
            return carry
        return body

    @pl.when(c == 0)
    def _():
        lax.fori_loop(0, AGG_CHUNKS, make_body(h0), 0)

    @pl.when(c == 1)
    def _():
        lax.fori_loop(0, AGG_CHUNKS, make_body(h1), 0)

    plsc.subcore_barrier()
    sl = pl.ds(base, ROWS_PER_TILE)

    @pl.when(c == 0)
    def _():
        pltpu.sync_copy(agg_s.at[sl], out0.at[sl])

    @pl.when(c == 1)
    def _():
        pltpu.sync_copy(agg_s.at[sl], out1.at[sl])


@functools.lru_cache(maxsize=None)
def _sc_calls():
    mesh = plsc.VectorSubcoreMesh(core_axis_name="c", subcore_axis_name="s",
                                  num_cores=NC, num_subcores=NS)
    deg = pl.kernel(
        _deg_body,
        out_type=jax.ShapeDtypeStruct((2, N_PAD, CHUNK), jnp.float32),
        mesh=mesh,
        scratch_types=[
            pltpu.VMEM((AGG_CHUNKS, CHUNK), jnp.int32),
            pltpu.VMEM((CHUNK, CHUNK), jnp.float32),
            pltpu.VMEM_SHARED((N_PAD, CHUNK), jnp.float32),
        ],
    )
    agg = pl.kernel(
        _agg_body,
        out_type=[jax.ShapeDtypeStruct((N_PAD, FH), jnp.float32),
                  jax.ShapeDtypeStruct((N_PAD, FH), jnp.float32)],
        mesh=mesh,
        scratch_types=[
            pltpu.VMEM((AGG_CHUNKS, CHUNK), jnp.int32),
            pltpu.VMEM((AGG_CHUNKS, CHUNK), jnp.int32),
            pltpu.VMEM((CHUNK, FH), jnp.float32),
            pltpu.VMEM_SHARED((N_PAD, FH), jnp.float32),
        ],
    )
    return deg, agg


_BR = 256


def _scale_out(deg_ref):
    return lax.rsqrt(jnp.maximum(deg_ref[0, :, 0], 1.0)).reshape(-1, 1)


def _scale_in(deg_ref):
    return lax.rsqrt(jnp.maximum(deg_ref[1, :, 0], 1.0)).reshape(-1, 1)


def _b1_kernel(x_ref, w_ref, deg_ref, out0_ref, out1_ref):
    h = jnp.dot(x_ref[:], w_ref[:], preferred_element_type=jnp.float32)
    h = h * _scale_out(deg_ref)
    out0_ref[:] = h[:, :FH]
    out1_ref[:] = h[:, FH:]


def _b2_kernel(a0_ref, a1_ref, deg_ref, b_ref, w_ref, out0_ref, out1_ref):
    s_i = _scale_in(deg_ref)
    s_o = _scale_out(deg_ref)
    y0 = jnp.maximum(a0_ref[:] * s_i + b_ref[0, :FH], 0.0) * s_o
    y1 = jnp.maximum(a1_ref[:] * s_i + b_ref[0, FH:], 0.0) * s_o
    h = (jnp.dot(y0, w_ref[:FH, :], preferred_element_type=jnp.float32)
         + jnp.dot(y1, w_ref[FH:, :], preferred_element_type=jnp.float32))
    out0_ref[:] = h[:, :FH]
    out1_ref[:] = h[:, FH:]


def _d_kernel(a0_ref, a1_ref, deg_ref, b_ref, wd_ref, bd_ref, out_ref, acc_ref):
    i = pl.program_id(0)

    @pl.when(i == 0)
    def _():
        acc_ref[:] = jnp.zeros_like(acc_ref)

    s_i = _scale_in(deg_ref)
    rows = i * _BR + lax.broadcasted_iota(jnp.int32, (_BR, 1), 0)
    mask = (rows < N_NODES).astype(jnp.float32)
    y0 = jnp.maximum(a0_ref[:] * s_i + b_ref[0, :FH], 0.0) * mask
    y1 = jnp.maximum(a1_ref[:] * s_i + b_ref[0, FH:], 0.0) * mask
    acc_ref[:, :FH] += jnp.sum(y0, axis=0, keepdims=True)
    acc_ref[:, FH:] += jnp.sum(y1, axis=0, keepdims=True)

    @pl.when(i == N_PAD // _BR - 1)
    def _():
        hg = acc_ref[:] * (1.0 / N_NODES)
        out_ref[:] = (jnp.dot(hg, wd_ref[:], preferred_element_type=jnp.float32)
                      + bd_ref[:])


@functools.lru_cache(maxsize=None)
def _tc_calls():
    nblk = N_PAD // _BR
    deg_spec = pl.BlockSpec((2, _BR, CHUNK), lambda i: (0, i, 0))
    half = pl.BlockSpec((_BR, FH), lambda i: (i, 0))
    full = pl.BlockSpec((_BR, F), lambda i: (i, 0))
    w_spec = pl.BlockSpec((F, F), lambda i: (0, 0))
    b_spec = pl.BlockSpec((1, F), lambda i: (0, 0))
    b1 = pl.pallas_call(
        _b1_kernel,
        grid=(nblk,),
        in_specs=[full, w_spec, deg_spec],
        out_specs=[half, half],
        out_shape=[jax.ShapeDtypeStruct((N_PAD, FH), jnp.float32)] * 2,
    )
    b2 = pl.pallas_call(
        _b2_kernel,
        grid=(nblk,),
        in_specs=[half, half, deg_spec, b_spec, w_spec],
        out_specs=[half, half],
        out_shape=[jax.ShapeDtypeStruct((N_PAD, FH), jnp.float32)] * 2,
    )
    d = pl.pallas_call(
        _d_kernel,
        grid=(nblk,),
        in_specs=[half, half, deg_spec, b_spec,
                  pl.BlockSpec((F, OUT_F), lambda i: (0, 0)),
                  pl.BlockSpec((1, OUT_F), lambda i: (0, 0))],
        out_specs=pl.BlockSpec((1, OUT_F), lambda i: (0, 0)),
        out_shape=jax.ShapeDtypeStruct((1, OUT_F), jnp.float32),
        scratch_shapes=[pltpu.VMEM((1, F), jnp.float32)],
    )
    return b1, b2, d


@jax.jit
def kernel(x, edge_index, W1, b1, W2, b2, Wd, bd):
    n_edges = edge_index.shape[1]
    src = edge_index[0].astype(jnp.int32)
    dst = edge_index[1].astype(jnp.int32)
    pad = jnp.full((E_PAD - n_edges,), N_NODES, jnp.int32)
    srcp = jnp.concatenate([src, pad]).reshape(NS, AGG_CHUNKS, CHUNK)
    dstp = jnp.concatenate([dst, pad]).reshape(NS, AGG_CHUNKS, CHUNK)
    xp = jnp.zeros((N_PAD, F), jnp.float32).at[:N_NODES].set(x)
    ones128 = jnp.ones((CHUNK, CHUNK), jnp.float32)
    zeros128 = jnp.zeros((ROWS_PER_TILE, FH), jnp.float32)

    deg_call, agg_call = _sc_calls()
    b1_call, b2_call, d_call = _tc_calls()

    degp = deg_call(srcp, dstp, ones128, zeros128)
    h10, h11 = b1_call(xp, W1, degp)
    a10, a11 = agg_call(h10, h11, srcp, dstp, zeros128)
    h20, h21 = b2_call(a10, a11, degp, b1.reshape(1, F), W2)
    a20, a21 = agg_call(h20, h21, srcp, dstp, zeros128)
    out = d_call(a20, a21, degp, b2.reshape(1, F), Wd, bd.reshape(1, OUT_F))
    return out


# speedup vs baseline: 3.3182x; 1.0943x over previous
import functools

import jax
import jax.numpy as jnp
from jax import lax
from jax.experimental import pallas as pl
from jax.experimental.pallas import tpu as pltpu
from jax.experimental.pallas import tpu_sc as plsc

N_NODES = 10000
N_PAD = 10240
F = 256
FH = 128
OUT_F = 128
NC, NS = 2, 16
CHUNK = 128
AGG_CHUNKS = 80
DEG_CHUNKS = 40
E_PAD = NS * AGG_CHUNKS * CHUNK
ROWS_PER_TILE = N_PAD // NS
DEG_W = 16


def _deg_body(srcp, dstp, ones_hbm, zeros_hbm, out_hbm, idx_v, const_v, deg_s):
    c = lax.axis_index("c")
    s = lax.axis_index("s")
    base = s * ROWS_PER_TILE
    pltpu.sync_copy(zeros_hbm, const_v)
    for k in range(ROWS_PER_TILE // CHUNK):
        pltpu.sync_copy(const_v, deg_s.at[pl.ds(base + k * CHUNK, CHUNK)])

    @pl.when(c == 0)
    def _():
        pltpu.sync_copy(srcp.at[s], idx_v)

    @pl.when(c == 1)
    def _():
        pltpu.sync_copy(dstp.at[s], idx_v)

    pltpu.sync_copy(ones_hbm, const_v)
    plsc.subcore_barrier()

    def body(j, carry):
        pltpu.sync_copy(const_v, deg_s.at[idx_v.at[j]], add=True)
        return carry

    lax.fori_loop(0, AGG_CHUNKS, body, 0)
    plsc.subcore_barrier()
    for k in range(ROWS_PER_TILE // CHUNK):
        sl = pl.ds(base + k * CHUNK, CHUNK)
        pltpu.sync_copy(deg_s.at[sl], const_v)

        @pl.when(c == 0)
        def _():
            pltpu.sync_copy(const_v, out_hbm.at[0, sl])

        @pl.when(c == 1)
        def _():
            pltpu.sync_copy(const_v, out_hbm.at[1, sl])


def _agg_body(h0, h1, srcp, dstp, zeros_hbm, out0, out1,
              src_v, dst_v, rows_v, agg_s):
    c = lax.axis_index("c")
    s = lax.axis_index("s")
    base = s * ROWS_PER_TILE
    pltpu.sync_copy(zeros_hbm, rows_v)
    for k in range(ROWS_PER_TILE // CHUNK):
        pltpu.sync_copy(rows_v, agg_s.at[pl.ds(base + k * CHUNK, CHUNK)])
    pltpu.sync_copy(srcp.at[s], src_v)
    pltpu.sync_copy(dstp.at[s], dst_v)
    plsc.subcore_barrier()

    def make_body(h_ref):
        def body(j, carry):
            pltpu.sync_copy(h_ref.at[src_v.at[j]], rows_v)
            pltpu.sync_copy(rows_v, agg_s.at[dst_v.at[j]], add=True)
            return carry
        return body

    @pl.when(c == 0)
    def _():
        lax.fori_loop(0, AGG_CHUNKS, make_body(h0), 0)

    @pl.when(c == 1)
    def _():
        lax.fori_loop(0, AGG_CHUNKS, make_body(h1), 0)

    plsc.subcore_barrier()
    for k in range(ROWS_PER_TILE // CHUNK):
        sl = pl.ds(base + k * CHUNK, CHUNK)
        pltpu.sync_copy(agg_s.at[sl], rows_v)

        @pl.when(c == 0)
        def _():
            pltpu.sync_copy(rows_v, out0.at[sl])

        @pl.when(c == 1)
        def _():
            pltpu.sync_copy(rows_v, out1.at[sl])


@functools.lru_cache(maxsize=None)
def _sc_calls():
    mesh = plsc.VectorSubcoreMesh(core_axis_name="c", subcore_axis_name="s",
                                  num_cores=NC, num_subcores=NS)
    deg = pl.kernel(
        _deg_body,
        out_type=jax.ShapeDtypeStruct((2, N_PAD, CHUNK), jnp.float32),
        mesh=mesh,
        scratch_types=[
            pltpu.VMEM((AGG_CHUNKS, CHUNK), jnp.int32),
            pltpu.VMEM((CHUNK, CHUNK), jnp.float32),
            pltpu.VMEM_SHARED((N_PAD, CHUNK), jnp.float32),
        ],
    )
    agg = pl.kernel(
        _agg_body,
        out_type=[jax.ShapeDtypeStruct((N_PAD, FH), jnp.float32),
                  jax.ShapeDtypeStruct((N_PAD, FH), jnp.float32)],
        mesh=mesh,
        scratch_types=[
            pltpu.VMEM((AGG_CHUNKS, CHUNK), jnp.int32),
            pltpu.VMEM((AGG_CHUNKS, CHUNK), jnp.int32),
            pltpu.VMEM((CHUNK, FH), jnp.float32),
            pltpu.VMEM_SHARED((N_PAD, FH), jnp.float32),
        ],
    )
    return deg, agg


_BR = 256


def _scale_out(deg_ref):
    return lax.rsqrt(jnp.maximum(deg_ref[0, :, 0], 1.0)).reshape(-1, 1)


def _scale_in(deg_ref):
    return lax.rsqrt(jnp.maximum(deg_ref[1, :, 0], 1.0)).reshape(-1, 1)


def _mm_kernel(x_ref, w_ref, out0_ref, out1_ref):
    h = jnp.dot(x_ref[:], w_ref[:], preferred_element_type=jnp.float32)
    out0_ref[:] = h[:, :FH]
    out1_ref[:] = h[:, FH:]


def _sc1_kernel(a0_ref, a1_ref, deg_ref, out0_ref, out1_ref):
    s_o = _scale_out(deg_ref)
    out0_ref[:] = a0_ref[:] * s_o
    out1_ref[:] = a1_ref[:] * s_o


def _b2_kernel(a0_ref, a1_ref, deg_ref, b_ref, w_ref, out0_ref, out1_ref):
    s_i = _scale_in(deg_ref)
    s_o = _scale_out(deg_ref)
    y0 = jnp.maximum(a0_ref[:] * s_i + b_ref[0, :FH], 0.0) * s_o
    y1 = jnp.maximum(a1_ref[:] * s_i + b_ref[0, FH:], 0.0) * s_o
    h = (jnp.dot(y0, w_ref[:FH, :], preferred_element_type=jnp.float32)
         + jnp.dot(y1, w_ref[FH:, :], preferred_element_type=jnp.float32))
    out0_ref[:] = h[:, :FH]
    out1_ref[:] = h[:, FH:]


def _d_kernel(a0_ref, a1_ref, deg_ref, b_ref, wd_ref, bd_ref, out_ref, acc_ref):
    i = pl.program_id(0)

    @pl.when(i == 0)
    def _():
        acc_ref[:] = jnp.zeros_like(acc_ref)

    s_i = _scale_in(deg_ref)
    rows = i * _BR + lax.broadcasted_iota(jnp.int32, (_BR, 1), 0)
    mask = (rows < N_NODES).astype(jnp.float32)
    y0 = jnp.maximum(a0_ref[:] * s_i + b_ref[0, :FH], 0.0) * mask
    y1 = jnp.maximum(a1_ref[:] * s_i + b_ref[0, FH:], 0.0) * mask
    acc_ref[:, :FH] += jnp.sum(y0, axis=0, keepdims=True)
    acc_ref[:, FH:] += jnp.sum(y1, axis=0, keepdims=True)

    @pl.when(i == N_PAD // _BR - 1)
    def _():
        hg = acc_ref[:] * (1.0 / N_NODES)
        out_ref[:] = (jnp.dot(hg, wd_ref[:], preferred_element_type=jnp.float32)
                      + bd_ref[:])


@functools.lru_cache(maxsize=None)
def _tc_calls():
    nblk = N_PAD // _BR
    deg_spec = pl.BlockSpec((2, _BR, CHUNK), lambda i: (0, i, 0))
    half = pl.BlockSpec((_BR, FH), lambda i: (i, 0))
    full = pl.BlockSpec((_BR, F), lambda i: (i, 0))
    w_spec = pl.BlockSpec((F, F), lambda i: (0, 0))
    b_spec = pl.BlockSpec((1, F), lambda i: (0, 0))
    mm = pl.pallas_call(
        _mm_kernel,
        grid=(nblk,),
        in_specs=[full, w_spec],
        out_specs=[half, half],
        out_shape=[jax.ShapeDtypeStruct((N_PAD, FH), jnp.float32)] * 2,
    )
    sc1 = pl.pallas_call(
        _sc1_kernel,
        grid=(nblk,),
        in_specs=[half, half, deg_spec],
        out_specs=[half, half],
        out_shape=[jax.ShapeDtypeStruct((N_PAD, FH), jnp.float32)] * 2,
    )
    b2 = pl.pallas_call(
        _b2_kernel,
        grid=(nblk,),
        in_specs=[half, half, deg_spec, b_spec, w_spec],
        out_specs=[half, half],
        out_shape=[jax.ShapeDtypeStruct((N_PAD, FH), jnp.float32)] * 2,
    )
    d = pl.pallas_call(
        _d_kernel,
        grid=(nblk,),
        in_specs=[half, half, deg_spec, b_spec,
                  pl.BlockSpec((F, OUT_F), lambda i: (0, 0)),
                  pl.BlockSpec((1, OUT_F), lambda i: (0, 0))],
        out_specs=pl.BlockSpec((1, OUT_F), lambda i: (0, 0)),
        out_shape=jax.ShapeDtypeStruct((1, OUT_F), jnp.float32),
        scratch_shapes=[pltpu.VMEM((1, F), jnp.float32)],
    )
    return mm, sc1, b2, d


@jax.jit
def kernel(x, edge_index, W1, b1, W2, b2, Wd, bd):
    n_edges = edge_index.shape[1]
    src = edge_index[0].astype(jnp.int32)
    dst = edge_index[1].astype(jnp.int32)
    pad = jnp.full((E_PAD - n_edges,), N_NODES, jnp.int32)
    srcp = jnp.concatenate([src, pad]).reshape(NS, AGG_CHUNKS, CHUNK)
    dstp = jnp.concatenate([dst, pad]).reshape(NS, AGG_CHUNKS, CHUNK)
    xp = jnp.zeros((N_PAD, F), jnp.float32).at[:N_NODES].set(x)
    ones128 = jnp.ones((CHUNK, CHUNK), jnp.float32)
    zeros128 = jnp.zeros((CHUNK, FH), jnp.float32)

    deg_call, agg_call = _sc_calls()
    mm_call, sc1_call, b2_call, d_call = _tc_calls()

    xw0, xw1 = mm_call(xp, W1)
    degp = deg_call(srcp, dstp, ones128, zeros128)
    h10, h11 = sc1_call(xw0, xw1, degp)
    a10, a11 = agg_call(h10, h11, srcp, dstp, zeros128)
    h20, h21 = b2_call(a10, a11, degp, b1.reshape(1, F), W2)
    a20, a21 = agg_call(h20, h21, srcp, dstp, zeros128)
    out = d_call(a20, a21, degp, b2.reshape(1, F), Wd, bd.reshape(1, OUT_F))
    return out


# speedup vs baseline: 3.3696x; 1.0155x over previous
import functools

import jax
import jax.numpy as jnp
from jax import lax
from jax.experimental import pallas as pl
from jax.experimental.pallas import tpu as pltpu
from jax.experimental.pallas import tpu_sc as plsc

N_NODES = 10000
N_PAD = 10240
F = 256
FH = 128
OUT_F = 128
NC, NS = 2, 16
CHUNK = 128
AGG_CHUNKS = 80
DEG_CHUNKS = 40
E_PAD = NS * AGG_CHUNKS * CHUNK
ROWS_PER_TILE = N_PAD // NS
DEG_W = 16


def _deg_body(srcp, dstp, ones_hbm, zeros_hbm, out_hbm, idx_v, const_v, deg_s):
    c = lax.axis_index("c")
    s = lax.axis_index("s")
    base = s * ROWS_PER_TILE
    pltpu.sync_copy(zeros_hbm, const_v)
    for k in range(ROWS_PER_TILE // CHUNK):
        pltpu.sync_copy(const_v, deg_s.at[pl.ds(base + k * CHUNK, CHUNK)])

    @pl.when(c == 0)
    def _():
        pltpu.sync_copy(srcp.at[s], idx_v)

    @pl.when(c == 1)
    def _():
        pltpu.sync_copy(dstp.at[s], idx_v)

    pltpu.sync_copy(ones_hbm, const_v)
    plsc.subcore_barrier()

    def body(j, carry):
        pltpu.sync_copy(const_v, deg_s.at[idx_v.at[j]], add=True)
        return carry

    lax.fori_loop(0, AGG_CHUNKS, body, 0)
    plsc.subcore_barrier()
    for k in range(ROWS_PER_TILE // CHUNK):
        sl = pl.ds(base + k * CHUNK, CHUNK)
        pltpu.sync_copy(deg_s.at[sl], const_v)

        @pl.when(c == 0)
        def _():
            pltpu.sync_copy(const_v, out_hbm.at[0, sl])

        @pl.when(c == 1)
        def _():
            pltpu.sync_copy(const_v, out_hbm.at[1, sl])


def _agg_body(h0, h1, srcp, dstp, zeros_hbm, out0, out1,
              src_v, dst_v, rows_v, agg_s):
    c = lax.axis_index("c")
    s = lax.axis_index("s")
    base = s * ROWS_PER_TILE
    pltpu.sync_copy(zeros_hbm, rows_v)
    for k in range(ROWS_PER_TILE // CHUNK):
        pltpu.sync_copy(rows_v, agg_s.at[pl.ds(base + k * CHUNK, CHUNK)])
    pltpu.sync_copy(srcp.at[s], src_v)
    pltpu.sync_copy(dstp.at[s], dst_v)
    plsc.subcore_barrier()

    def make_body(h_ref):
        def body(j, carry):
            pltpu.sync_copy(h_ref.at[src_v.at[j]], rows_v)
            pltpu.sync_copy(rows_v, agg_s.at[dst_v.at[j]], add=True)
            return carry
        return body

    @pl.when(c == 0)
    def _():
        lax.fori_loop(0, AGG_CHUNKS, make_body(h0), 0)

    @pl.when(c == 1)
    def _():
        lax.fori_loop(0, AGG_CHUNKS, make_body(h1), 0)

    plsc.subcore_barrier()
    for k in range(ROWS_PER_TILE // CHUNK):
        sl = pl.ds(base + k * CHUNK, CHUNK)
        pltpu.sync_copy(agg_s.at[sl], rows_v)

        @pl.when(c == 0)
        def _():
            pltpu.sync_copy(rows_v, out0.at[sl])

        @pl.when(c == 1)
        def _():
            pltpu.sync_copy(rows_v, out1.at[sl])


@functools.lru_cache(maxsize=None)
def _sc_calls():
    mesh = plsc.VectorSubcoreMesh(core_axis_name="c", subcore_axis_name="s",
                                  num_cores=NC, num_subcores=NS)
    deg = pl.kernel(
        _deg_body,
        out_type=jax.ShapeDtypeStruct((2, N_PAD, CHUNK), jnp.float32),
        mesh=mesh,
        scratch_types=[
            pltpu.VMEM((AGG_CHUNKS, CHUNK), jnp.int32),
            pltpu.VMEM((CHUNK, CHUNK), jnp.float32),
            pltpu.VMEM_SHARED((N_PAD, CHUNK), jnp.float32),
        ],
    )
    agg = pl.kernel(
        _agg_body,
        out_type=[jax.ShapeDtypeStruct((N_PAD, FH), jnp.float32),
                  jax.ShapeDtypeStruct((N_PAD, FH), jnp.float32)],
        mesh=mesh,
        scratch_types=[
            pltpu.VMEM((AGG_CHUNKS, CHUNK), jnp.int32),
            pltpu.VMEM((AGG_CHUNKS, CHUNK), jnp.int32),
            pltpu.VMEM((CHUNK, FH), jnp.float32),
            pltpu.VMEM_SHARED((N_PAD, FH), jnp.float32),
        ],
    )
    return deg, agg


_BR = 256


def _scale_out(deg_ref):
    return lax.rsqrt(jnp.maximum(deg_ref[0, :, 0], 1.0)).reshape(-1, 1)


def _scale_in(deg_ref):
    return lax.rsqrt(jnp.maximum(deg_ref[1, :, 0], 1.0)).reshape(-1, 1)


def _b1_kernel(x_ref, w_ref, deg_ref, out0_ref, out1_ref):
    h = jnp.dot(x_ref[:], w_ref[:], preferred_element_type=jnp.float32)
    h = h * _scale_out(deg_ref)
    out0_ref[:] = h[:, :FH]
    out1_ref[:] = h[:, FH:]


def _b2_kernel(a0_ref, a1_ref, deg_ref, b_ref, w_ref, out0_ref, out1_ref):
    s_i = _scale_in(deg_ref)
    s_o = _scale_out(deg_ref)
    y0 = jnp.maximum(a0_ref[:] * s_i + b_ref[0, :FH], 0.0) * s_o
    y1 = jnp.maximum(a1_ref[:] * s_i + b_ref[0, FH:], 0.0) * s_o
    h = (jnp.dot(y0, w_ref[:FH, :], preferred_element_type=jnp.float32)
         + jnp.dot(y1, w_ref[FH:, :], preferred_element_type=jnp.float32))
    out0_ref[:] = h[:, :FH]
    out1_ref[:] = h[:, FH:]


def _d_kernel(a0_ref, a1_ref, deg_ref, b_ref, wd_ref, bd_ref, out_ref, acc_ref):
    i = pl.program_id(0)

    @pl.when(i == 0)
    def _():
        acc_ref[:] = jnp.zeros_like(acc_ref)

    s_i = _scale_in(deg_ref)
    rows = i * _BR + lax.broadcasted_iota(jnp.int32, (_BR, 1), 0)
    mask = (rows < N_NODES).astype(jnp.float32)
    y0 = jnp.maximum(a0_ref[:] * s_i + b_ref[0, :FH], 0.0) * mask
    y1 = jnp.maximum(a1_ref[:] * s_i + b_ref[0, FH:], 0.0) * mask
    acc_ref[:, :FH] += jnp.sum(y0, axis=0, keepdims=True)
    acc_ref[:, FH:] += jnp.sum(y1, axis=0, keepdims=True)

    @pl.when(i == N_PAD // _BR - 1)
    def _():
        hg = acc_ref[:] * (1.0 / N_NODES)
        out_ref[:] = (jnp.dot(hg, wd_ref[:], preferred_element_type=jnp.float32)
                      + bd_ref[:])


@functools.lru_cache(maxsize=None)
def _tc_calls():
    nblk = N_PAD // _BR
    deg_spec = pl.BlockSpec((2, _BR, CHUNK), lambda i: (0, i, 0))
    half = pl.BlockSpec((_BR, FH), lambda i: (i, 0))
    full = pl.BlockSpec((_BR, F), lambda i: (i, 0))
    w_spec = pl.BlockSpec((F, F), lambda i: (0, 0))
    b_spec = pl.BlockSpec((1, F), lambda i: (0, 0))
    b1 = pl.pallas_call(
        _b1_kernel,
        grid=(nblk,),
        in_specs=[full, w_spec, deg_spec],
        out_specs=[half, half],
        out_shape=[jax.ShapeDtypeStruct((N_PAD, FH), jnp.float32)] * 2,
    )
    b2 = pl.pallas_call(
        _b2_kernel,
        grid=(nblk,),
        in_specs=[half, half, deg_spec, b_spec, w_spec],
        out_specs=[half, half],
        out_shape=[jax.ShapeDtypeStruct((N_PAD, FH), jnp.float32)] * 2,
    )
    d = pl.pallas_call(
        _d_kernel,
        grid=(nblk,),
        in_specs=[half, half, deg_spec, b_spec,
                  pl.BlockSpec((F, OUT_F), lambda i: (0, 0)),
                  pl.BlockSpec((1, OUT_F), lambda i: (0, 0))],
        out_specs=pl.BlockSpec((1, OUT_F), lambda i: (0, 0)),
        out_shape=jax.ShapeDtypeStruct((1, OUT_F), jnp.float32),
        scratch_shapes=[pltpu.VMEM((1, F), jnp.float32)],
    )
    return b1, b2, d


@jax.jit
def kernel(x, edge_index, W1, b1, W2, b2, Wd, bd):
    n_edges = edge_index.shape[1]
    src = edge_index[0].astype(jnp.int32)
    dst = edge_index[1].astype(jnp.int32)
    pad = jnp.full((E_PAD - n_edges,), N_NODES, jnp.int32)
    srcp = jnp.concatenate([src, pad]).reshape(NS, AGG_CHUNKS, CHUNK)
    dstp = jnp.concatenate([dst, pad]).reshape(NS, AGG_CHUNKS, CHUNK)
    xp = jnp.zeros((N_PAD, F), jnp.float32).at[:N_NODES].set(x)
    ones128 = jnp.ones((CHUNK, CHUNK), jnp.float32)
    zeros128 = jnp.zeros((CHUNK, FH), jnp.float32)

    deg_call, agg_call = _sc_calls()
    b1_call, b2_call, d_call = _tc_calls()

    degp = deg_call(srcp, dstp, ones128, zeros128)
    h10, h11 = b1_call(xp, W1, degp)
    a10, a11 = agg_call(h10, h11, srcp, dstp, zeros128)
    h20, h21 = b2_call(a10, a11, degp, b1.reshape(1, F), W2)
    a20, a21 = agg_call(h20, h21, srcp, dstp, zeros128)
    out = d_call(a20, a21, degp, b2.reshape(1, F), Wd, bd.reshape(1, OUT_F))
    return out
